# baseline (speedup 1.0000x reference)
import functools

import jax
import jax.numpy as jnp
from jax import lax
from jax.experimental import pallas as pl
from jax.experimental.pallas import tpu as pltpu
from jax.experimental.pallas import tpu_sc as plsc

NC = 2
NS = 16
NW = NC * NS
D = 32
LANES = 16
G = 128
NBUF = 8

_MESH = dict(core_axis_name="c", subcore_axis_name="s")
_CP = pltpu.CompilerParams(use_tc_tiling_on_sc=False)


def _worker_ids():
  c = lax.axis_index("c")
  s = lax.axis_index("s")
  return s * NC + c, s


def _shift_rowmap(sidx_v, ng, slab):
  @pl.loop(0, ng)
  def _(g):
    for j in range(0, G, LANES):
      sidx_v[g, pl.ds(j, LANES)] = sidx_v[g, pl.ds(j, LANES)] + slab


def _zero_slab(zeros_v, BW, acc_sh, slab):
  zero = jnp.zeros((LANES,), jnp.float32)

  @pl.loop(0, BW)
  def _(r):
    zeros_v[r, pl.ds(0, LANES)] = zero
    zeros_v[r, pl.ds(LANES, LANES)] = zero

  pltpu.sync_copy(zeros_v, acc_sh.at[pl.ds(slab, BW)])


def _bag_phase(tab_hbm, idx_v, sidx_v, ng, acc, bufs, gsems, ssems):
  for b in range(min(NBUF, ng)):
    pltpu.async_copy(tab_hbm.at[idx_v.at[b]], bufs[b], gsems[b])

  @pl.loop(0, (ng + NBUF - 1) // NBUF)
  def _(h):
    g0 = h * NBUF
    for b in range(NBUF):
      g = g0 + b

      @pl.when(g < ng)
      def _():
        pltpu.make_async_copy(
            tab_hbm.at[idx_v.at[g]], bufs[b], gsems[b]).wait()
        pltpu.async_copy(bufs[b], acc.at[sidx_v.at[g]], ssems[b], add=True)

        @pl.when(g + NBUF < ng)
        def _():
          pltpu.make_async_copy(
              bufs[b], acc.at[sidx_v.at[g]], ssems[b]).wait()
          pltpu.async_copy(tab_hbm.at[idx_v.at[g + NBUF]], bufs[b], gsems[b])

  for b in range(min(NBUF, ng)):
    pltpu.make_async_copy(bufs[b], acc.at[sidx_v.at[0]], ssems[b]).wait()


def _build_small_kernel(B, n_small):
  BW = B // NW

  @functools.partial(
      pl.kernel,
      out_type=(jax.ShapeDtypeStruct((B, D), jnp.float32),
                jax.ShapeDtypeStruct((B, D), jnp.float32)),
      mesh=plsc.VectorSubcoreMesh(**_MESH),
      compiler_params=_CP,
      scratch_types=[
          pltpu.VMEM((n_small, G), jnp.int32),
          pltpu.VMEM((n_small, G), jnp.int32),
          pltpu.VMEM((G,), jnp.int32),
          [pltpu.VMEM((G, D), jnp.float32)] * NBUF,
          [pltpu.SemaphoreType.DMA] * NBUF,
          [pltpu.SemaphoreType.DMA] * NBUF,
          pltpu.VMEM((BW, D), jnp.float32),
          pltpu.VMEM_SHARED((NS * BW, D), jnp.float32),
      ],
  )
  def k(small_idx, users_idx, maps_small, s_tab, u_tab, out_c, out_u,
        idx_v, sidx_v, idx_users_v, bufs, gsems, ssems, zeros_v, acc_sh):
    wid, s = _worker_ids()
    base = wid * BW
    slab = s * BW
    pltpu.sync_copy(small_idx.at[wid], idx_v)
    pltpu.sync_copy(users_idx.at[wid], idx_users_v)
    pltpu.sync_copy(maps_small, sidx_v)
    _shift_rowmap(sidx_v, n_small, slab)
    _zero_slab(zeros_v, BW, acc_sh, slab)
    _bag_phase(s_tab, idx_v, sidx_v, n_small, acc_sh, bufs, gsems, ssems)
    pltpu.sync_copy(acc_sh.at[pl.ds(slab, BW)], out_c.at[pl.ds(base, BW)])
    pltpu.async_copy(u_tab.at[idx_users_v], bufs[0], gsems[0]).wait()
    pltpu.sync_copy(bufs[0], out_u.at[pl.ds(base, BW)])

  return k


def _build_items_kernel(B, n_items):
  BW = B // NW

  @functools.partial(
      pl.kernel,
      out_type=jax.ShapeDtypeStruct((B, D), jnp.float32),
      mesh=plsc.VectorSubcoreMesh(**_MESH),
      compiler_params=_CP,
      scratch_types=[
          pltpu.VMEM((n_items, G), jnp.int32),
          pltpu.VMEM((n_items, G), jnp.int32),
          [pltpu.VMEM((G, D), jnp.float32)] * NBUF,
          [pltpu.SemaphoreType.DMA] * NBUF,
          [pltpu.SemaphoreType.DMA] * NBUF,
          pltpu.VMEM((BW, D), jnp.float32),
          pltpu.VMEM_SHARED((NS * BW, D), jnp.float32),
      ],
  )
  def k(items_idx, maps_items, i_tab, out_i,
        idx_v, sidx_v, bufs, gsems, ssems, zeros_v, acc_sh):
    wid, s = _worker_ids()
    base = wid * BW
    slab = s * BW
    pltpu.sync_copy(items_idx.at[wid], idx_v)
    pltpu.sync_copy(maps_items, sidx_v)
    _shift_rowmap(sidx_v, n_items, slab)
    _zero_slab(zeros_v, BW, acc_sh, slab)
    _bag_phase(i_tab, idx_v, sidx_v, n_items, acc_sh, bufs, gsems, ssems)
    pltpu.sync_copy(acc_sh.at[pl.ds(slab, BW)], out_i.at[pl.ds(base, BW)])

  return k


def _tr_body(x_ref, o_ref):
  y = x_ref[...].T
  pk = o_ref.shape[0]
  z = y.reshape(pk, 128 // y.shape[1], y.shape[1])
  o_ref[...] = jnp.concatenate(
      [z[:, q, :] for q in range(z.shape[1])], axis=1)


def _to_linear_rowmajor(w, blk=8192):
  n, d = w.shape
  t = jnp.swapaxes(w, 0, 1)
  grid = (n + blk - 1) // blk
  pk = blk * d // 128
  out = pl.pallas_call(
      _tr_body,
      grid=(grid,),
      in_specs=[pl.BlockSpec((d, blk), lambda i: (0, i))],
      out_specs=pl.BlockSpec((pk, 128), lambda i: (i, 0)),
      out_shape=jax.ShapeDtypeStruct((n * d // 128, 128), jnp.float32),
  )(t)
  return out.reshape(n, d)


def _dot_body(x_ref, y_ref, z_ref, o_ref):
  o_ref[...] = jnp.sum(x_ref[...] * (y_ref[...] + z_ref[...]), axis=1)


def _rowwise_dot(x, y, z):
  B = x.shape[0]
  return pl.pallas_call(
      _dot_body,
      out_shape=jax.ShapeDtypeStruct((B,), jnp.float32),
  )(x, y, z)


def kernel(cates, attrs, cate_context, attr_context, users, items,
           u_emb_w, i_emb_w, cate_w, attr_w, cate_c_w, attr_c_w):
  B = cates.shape[0]
  C = cate_w.shape[0]
  A = attr_w.shape[0]
  CC = cate_c_w.shape[0]
  BW = B // NW

  small_tab = jnp.concatenate([
      cate_w.at[-1].set(0.0),
      attr_w.at[-1].set(0.0),
      cate_c_w.at[-1].set(0.0),
      attr_c_w.at[-1].set(0.0),
  ], axis=0)
  small_idx = jnp.concatenate([
      cates,
      attrs + C,
      cate_context + C + A,
      attr_context + C + A + CC,
  ], axis=1)

  k_items = items.shape[1]
  k_small = small_idx.shape[1]
  n_items = (BW * k_items) // G
  n_small = (BW * k_small) // G

  items_b = items.astype(jnp.int32).reshape(NW, n_items, G)
  small_b = small_idx.astype(jnp.int32).reshape(NW, n_small, G)
  users_b = users.reshape(NW, G).astype(jnp.int32)

  maps_items = (jnp.arange(n_items * G, dtype=jnp.int32) // k_items
                ).reshape(n_items, G)
  maps_small = (jnp.arange(n_small * G, dtype=jnp.int32) // k_small
                ).reshape(n_small, G)

  u_tab = _to_linear_rowmajor(u_emb_w)
  small_b, users_b, maps_small, small_tab, u_tab, i_emb_w = (
      lax.optimization_barrier((small_b, users_b, maps_small, small_tab,
                                u_tab, i_emb_w)))

  acc_c, u_acc = _build_small_kernel(B, n_small)(
      small_b, users_b, maps_small, small_tab, u_tab)

  i_tab = _to_linear_rowmajor(i_emb_w)
  acc_i = _build_items_kernel(B, n_items)(items_b, maps_items, i_tab)
  return _rowwise_dot(acc_i, acc_c, u_acc)


# speedup vs baseline: 1.1292x; 1.1292x over previous
import functools

import jax
import jax.numpy as jnp
from jax import lax
from jax.experimental import pallas as pl
from jax.experimental.pallas import tpu as pltpu
from jax.experimental.pallas import tpu_sc as plsc

NC = 2
NS = 16
NW = NC * NS
D = 32
LANES = 16
G = 128
NBUF = 8

_MESH = dict(core_axis_name="c", subcore_axis_name="s")
_CP = pltpu.CompilerParams(use_tc_tiling_on_sc=False)


def _worker_ids():
  c = lax.axis_index("c")
  s = lax.axis_index("s")
  return s * NC + c, s


def _shift_rowmap(sidx_v, ng, slab):
  @pl.loop(0, ng)
  def _(g):
    for j in range(0, G, LANES):
      sidx_v[g, pl.ds(j, LANES)] = sidx_v[g, pl.ds(j, LANES)] + slab


def _zero_slab(zeros_v, BW, acc_sh, slab):
  zero = jnp.zeros((LANES,), jnp.float32)

  @pl.loop(0, BW)
  def _(r):
    zeros_v[r, pl.ds(0, LANES)] = zero
    zeros_v[r, pl.ds(LANES, LANES)] = zero

  pltpu.sync_copy(zeros_v, acc_sh.at[pl.ds(slab, BW)])


def _bag_phase(tab_hbm, idx_v, sidx_v, ng, acc, bufs, gsems, ssems):
  for b in range(min(NBUF, ng)):
    pltpu.async_copy(tab_hbm.at[idx_v.at[b]], bufs[b], gsems[b])

  @pl.loop(0, (ng + NBUF - 1) // NBUF)
  def _(h):
    g0 = h * NBUF
    for b in range(NBUF):
      g = g0 + b

      @pl.when(g < ng)
      def _():
        pltpu.make_async_copy(
            tab_hbm.at[idx_v.at[g]], bufs[b], gsems[b]).wait()
        pltpu.async_copy(bufs[b], acc.at[sidx_v.at[g]], ssems[b], add=True)

        @pl.when(g + NBUF < ng)
        def _():
          pltpu.make_async_copy(
              bufs[b], acc.at[sidx_v.at[g]], ssems[b]).wait()
          pltpu.async_copy(tab_hbm.at[idx_v.at[g + NBUF]], bufs[b], gsems[b])

  for b in range(min(NBUF, ng)):
    pltpu.make_async_copy(bufs[b], acc.at[sidx_v.at[0]], ssems[b]).wait()


def _build_small_kernel(B, n_small):
  BW = B // NW

  @functools.partial(
      pl.kernel,
      out_type=jax.ShapeDtypeStruct((B, D), jnp.float32),
      mesh=plsc.VectorSubcoreMesh(**_MESH),
      compiler_params=_CP,
      scratch_types=[
          pltpu.VMEM((n_small, G), jnp.int32),
          pltpu.VMEM((n_small, G), jnp.int32),
          [pltpu.VMEM((G, D), jnp.float32)] * NBUF,
          [pltpu.SemaphoreType.DMA] * NBUF,
          [pltpu.SemaphoreType.DMA] * NBUF,
          pltpu.VMEM((BW, D), jnp.float32),
          pltpu.VMEM_SHARED((NS * BW, D), jnp.float32),
      ],
  )
  def k(small_idx, maps_small, s_tab, out_c,
        idx_v, sidx_v, bufs, gsems, ssems, zeros_v, acc_sh):
    wid, s = _worker_ids()
    base = wid * BW
    slab = s * BW
    pltpu.sync_copy(small_idx.at[wid], idx_v)
    pltpu.sync_copy(maps_small, sidx_v)
    _shift_rowmap(sidx_v, n_small, slab)
    _zero_slab(zeros_v, BW, acc_sh, slab)
    _bag_phase(s_tab, idx_v, sidx_v, n_small, acc_sh, bufs, gsems, ssems)
    pltpu.sync_copy(acc_sh.at[pl.ds(slab, BW)], out_c.at[pl.ds(base, BW)])

  return k


def _build_items_kernel(B, n_items):
  BW = B // NW

  @functools.partial(
      pl.kernel,
      out_type=(jax.ShapeDtypeStruct((B, D), jnp.float32),
                jax.ShapeDtypeStruct((B, D), jnp.float32)),
      mesh=plsc.VectorSubcoreMesh(**_MESH),
      compiler_params=_CP,
      scratch_types=[
          pltpu.VMEM((n_items, G), jnp.int32),
          pltpu.VMEM((n_items, G), jnp.int32),
          pltpu.VMEM((G,), jnp.int32),
          [pltpu.VMEM((G, D), jnp.float32)] * NBUF,
          [pltpu.SemaphoreType.DMA] * NBUF,
          [pltpu.SemaphoreType.DMA] * NBUF,
          pltpu.VMEM((BW, D), jnp.float32),
          pltpu.VMEM_SHARED((NS * BW, D), jnp.float32),
      ],
  )
  def k(items_idx, users_idx, maps_items, i_tab, u_tab, out_i, out_u,
        idx_v, sidx_v, idx_users_v, bufs, gsems, ssems, zeros_v, acc_sh):
    wid, s = _worker_ids()
    base = wid * BW
    slab = s * BW
    pltpu.sync_copy(items_idx.at[wid], idx_v)
    pltpu.sync_copy(users_idx.at[wid], idx_users_v)
    pltpu.sync_copy(maps_items, sidx_v)
    _shift_rowmap(sidx_v, n_items, slab)
    _zero_slab(zeros_v, BW, acc_sh, slab)
    _bag_phase(i_tab, idx_v, sidx_v, n_items, acc_sh, bufs, gsems, ssems)
    pltpu.sync_copy(acc_sh.at[pl.ds(slab, BW)], out_i.at[pl.ds(base, BW)])
    pltpu.async_copy(u_tab.at[idx_users_v], bufs[0], gsems[0]).wait()
    pltpu.sync_copy(bufs[0], out_u.at[pl.ds(base, BW)])

  return k


def _tr_body(x_ref, o_ref):
  y = x_ref[...].T
  pk = o_ref.shape[0]
  z = y.reshape(pk, 128 // y.shape[1], y.shape[1])
  o_ref[...] = jnp.concatenate(
      [z[:, q, :] for q in range(z.shape[1])], axis=1)


def _to_linear_rowmajor(w, blk=8192):
  n, d = w.shape
  t = jnp.swapaxes(w, 0, 1)
  grid = (n + blk - 1) // blk
  pk = blk * d // 128
  out = pl.pallas_call(
      _tr_body,
      grid=(grid,),
      in_specs=[pl.BlockSpec((d, blk), lambda i: (0, i))],
      out_specs=pl.BlockSpec((pk, 128), lambda i: (i, 0)),
      out_shape=jax.ShapeDtypeStruct((n * d // 128, 128), jnp.float32),
  )(t)
  return out.reshape(n, d)


def _dot_body(x_ref, y_ref, z_ref, o_ref):
  o_ref[...] = jnp.sum(x_ref[...] * (y_ref[...] + z_ref[...]), axis=1)


def _rowwise_dot(x, y, z):
  B = x.shape[0]
  return pl.pallas_call(
      _dot_body,
      out_shape=jax.ShapeDtypeStruct((B,), jnp.float32),
  )(x, y, z)


def kernel(cates, attrs, cate_context, attr_context, users, items,
           u_emb_w, i_emb_w, cate_w, attr_w, cate_c_w, attr_c_w):
  B = cates.shape[0]
  C = cate_w.shape[0]
  A = attr_w.shape[0]
  CC = cate_c_w.shape[0]
  BW = B // NW

  small_tab = jnp.concatenate([
      cate_w.at[-1].set(0.0),
      attr_w.at[-1].set(0.0),
      cate_c_w.at[-1].set(0.0),
      attr_c_w.at[-1].set(0.0),
  ], axis=0)
  small_idx = jnp.concatenate([
      cates,
      attrs + C,
      cate_context + C + A,
      attr_context + C + A + CC,
  ], axis=1)

  k_items = items.shape[1]
  k_small = small_idx.shape[1]
  n_items = (BW * k_items) // G
  n_small = (BW * k_small) // G

  items_b = items.astype(jnp.int32).reshape(NW, n_items, G)
  small_b = small_idx.astype(jnp.int32).reshape(NW, n_small, G)
  users_b = users.reshape(NW, G).astype(jnp.int32)

  maps_items = (jnp.arange(n_items * G, dtype=jnp.int32) // k_items
                ).reshape(n_items, G)
  maps_small = (jnp.arange(n_small * G, dtype=jnp.int32) // k_small
                ).reshape(n_small, G)

  small_b, maps_small, small_tab, i_emb_w, u_emb_w = (
      lax.optimization_barrier((small_b, maps_small, small_tab,
                                i_emb_w, u_emb_w)))

  acc_c = _build_small_kernel(B, n_small)(small_b, maps_small, small_tab)

  i_tab = _to_linear_rowmajor(i_emb_w)
  u_tab = _to_linear_rowmajor(u_emb_w)
  acc_i, u_acc = _build_items_kernel(B, n_items)(
      items_b, users_b, maps_items, i_tab, u_tab)
  return _rowwise_dot(acc_i, acc_c, u_acc)
